# Initial kernel scaffold; baseline (speedup 1.0000x reference)
import jax
import jax.numpy as jnp
from jax.experimental import pallas as pl


def kernel(x, pos_table, gamma, beta):
    raise NotImplementedError("write your pallas kernel here")


# speedup vs baseline: 3.5130x; 3.5130x over previous
import jax
import jax.numpy as jnp
from jax.experimental import pallas as pl

_BLK_S = 512
_EPS = 1e-5


def _ln_body(x_ref, pos_ref, g_ref, b_ref, o_ref):
    h = x_ref[0] + pos_ref[...]
    mean = jnp.mean(h, axis=-1, keepdims=True)
    d = h - mean
    var = jnp.mean(d * d, axis=-1, keepdims=True)
    o_ref[0] = d * jax.lax.rsqrt(var + _EPS) * g_ref[...] + b_ref[...]


def kernel(x, pos_table, gamma, beta):
    B, S, D = x.shape
    gamma2 = gamma.reshape(1, D)
    beta2 = beta.reshape(1, D)
    grid = (S // _BLK_S, B)
    return pl.pallas_call(
        _ln_body,
        grid=grid,
        in_specs=[
            pl.BlockSpec((1, _BLK_S, D), lambda s, b: (b, s, 0)),
            pl.BlockSpec((_BLK_S, D), lambda s, b: (s, 0)),
            pl.BlockSpec((1, D), lambda s, b: (0, 0)),
            pl.BlockSpec((1, D), lambda s, b: (0, 0)),
        ],
        out_specs=pl.BlockSpec((1, _BLK_S, D), lambda s, b: (b, s, 0)),
        out_shape=jax.ShapeDtypeStruct((B, S, D), x.dtype),
    )(x, pos_table, gamma2, beta2)
